# baseline (speedup 1.0000x reference)
import functools

import jax
import jax.numpy as jnp
from jax import lax
from jax.experimental import pallas as pl
from jax.experimental.pallas import tpu as pltpu
from jax.experimental.pallas import tpu_sc as plsc

N, S, D, K = 4096, 200, 64, 50
NCS = 2
JPC = K // NCS

_mesh = plsc.ScalarSubcoreMesh(axis_name="c")


@functools.partial(
    pl.kernel,
    mesh=_mesh,
    out_type=jax.ShapeDtypeStruct((K, D, N), jnp.float32),
    scratch_types=[
        pltpu.SMEM((K,), jnp.int32),
        pltpu.SemaphoreType.DMA,
    ],
)
def _sc_gather(in_t, ind_hbm, out_t, ind_sm, sem):
    cid = lax.axis_index("c")
    j0 = cid * JPC
    pltpu.sync_copy(ind_hbm, ind_sm)

    def fire(j, carry):
        m = lax.rem(lax.rem(ind_sm[j], S) + S, S)
        pltpu.make_async_copy(in_t.at[m], out_t.at[j], sem).start()
        return carry

    lax.fori_loop(j0, j0 + JPC, fire, 0)

    def drain(j, carry):
        pltpu.make_async_copy(in_t.at[0], out_t.at[j], sem).wait()
        return carry

    lax.fori_loop(j0, j0 + JPC, drain, 0)


def kernel(inputs, indices):
    in_t = jnp.transpose(inputs, (1, 2, 0))
    out_t = _sc_gather(in_t, indices.astype(jnp.int32))
    return jnp.transpose(out_t, (2, 0, 1))


# speedup vs baseline: 26.8982x; 26.8982x over previous
import functools

import jax
import jax.numpy as jnp
from jax import lax
from jax.experimental import pallas as pl
from jax.experimental.pallas import tpu as pltpu
from jax.experimental.pallas import tpu_sc as plsc

N, S, D, K = 4096, 200, 64, 50
NC, NS = 2, 16
NW = NC * NS
CW = N // NW
CH = 8
DH = D // 2
NBUF = 3

_TASKS = []
for _c in range(4):
    for _h in range(2):
        _j0 = _c * 16 + _h * CH
        if _j0 >= K:
            continue
        _n = min(CH, K - _j0)
        for _dh in range(2):
            _TASKS.append((_c, _h, _dh, _j0, _n))

_mesh = plsc.VectorSubcoreMesh(core_axis_name="c", subcore_axis_name="s")


@functools.partial(
    pl.kernel,
    mesh=_mesh,
    out_type=jax.ShapeDtypeStruct((K, D, N), jnp.float32),
    scratch_types=[
        pltpu.VMEM((64,), jnp.int32),
        pltpu.VMEM((4, 16), jnp.int32),
        pltpu.VMEM((CH, DH, CW), jnp.float32),
        pltpu.VMEM((CH, DH, CW), jnp.float32),
        pltpu.VMEM((CH, DH, CW), jnp.float32),
        pltpu.SemaphoreType.DMA,
        pltpu.SemaphoreType.DMA,
    ],
)
def _sc_gather(in_t, ind_hbm, out_t, raw_v, idx_v, buf0, buf1, buf2, gsem, wsem):
    wid = lax.axis_index("s") * NC + lax.axis_index("c")
    c0 = wid * CW
    bufs = [buf0, buf1, buf2]

    pltpu.sync_copy(ind_hbm, raw_v.at[pl.ds(0, K)])
    for c in range(4):
        v = raw_v[pl.ds(c * 16, 16)]
        idx_v[c, pl.ds(0, 16)] = lax.rem(lax.rem(v, S) + S, S)

    def gather(t):
        c, h, dh, _, _ = _TASKS[t]
        pltpu.make_async_copy(
            in_t.at[idx_v.at[c, pl.ds(h * CH, CH)],
                    pl.ds(dh * DH, DH), pl.ds(c0, CW)],
            bufs[t % NBUF],
            gsem,
        ).start()

    def wait_gather(t):
        pltpu.make_async_copy(
            in_t.at[pl.ds(0, CH), pl.ds(0, DH), pl.ds(c0, CW)],
            bufs[t % NBUF],
            gsem,
        ).wait()

    def write(t):
        _, _, dh, j0, n = _TASKS[t]
        pltpu.make_async_copy(
            bufs[t % NBUF].at[pl.ds(0, n)],
            out_t.at[pl.ds(j0, n), pl.ds(dh * DH, DH), pl.ds(c0, CW)],
            wsem,
        ).start()

    def wait_write(t):
        _, _, _, _, n = _TASKS[t]
        pltpu.make_async_copy(
            bufs[t % NBUF].at[pl.ds(0, n)],
            out_t.at[pl.ds(0, n), pl.ds(0, DH), pl.ds(c0, CW)],
            wsem,
        ).wait()

    T = len(_TASKS)
    for t in range(min(NBUF - 1, T)):
        gather(t)
    waited = 0
    for t in range(T):
        k = t + NBUF - 1
        if k < T:
            if k - NBUF >= 0:
                wait_write(k - NBUF)
                waited = k - NBUF + 1
            gather(k)
        wait_gather(t)
        write(t)
    for t in range(waited, T):
        wait_write(t)


def kernel(inputs, indices):
    in_t = jnp.transpose(inputs, (1, 2, 0))
    out_t = _sc_gather(in_t, indices.astype(jnp.int32))
    return jnp.transpose(out_t, (2, 0, 1))


# speedup vs baseline: 28.3220x; 1.0529x over previous
import functools

import jax
import jax.numpy as jnp
from jax import lax
from jax.experimental import pallas as pl
from jax.experimental.pallas import tpu as pltpu
from jax.experimental.pallas import tpu_sc as plsc

N, S, D, K = 4096, 200, 64, 50
NC, NS = 2, 16
NW = NC * NS
CW = N // NW
CH = 8
DH = D // 2
NBUF = 3

_TASKS = []
for _c in range(4):
    for _h in range(2):
        _j0 = _c * 16 + _h * CH
        if _j0 >= K:
            continue
        _n = min(CH, K - _j0)
        for _dh in range(2):
            _TASKS.append((_c, _h, _dh, _j0, _n))

_mesh = plsc.VectorSubcoreMesh(core_axis_name="c", subcore_axis_name="s")


@functools.partial(
    pl.kernel,
    mesh=_mesh,
    out_type=jax.ShapeDtypeStruct((K, D, N), jnp.float32),
    scratch_types=[
        pltpu.VMEM((64,), jnp.int32),
        pltpu.VMEM((4, 16), jnp.int32),
        pltpu.VMEM((CH, DH, CW), jnp.float32),
        pltpu.VMEM((CH, DH, CW), jnp.float32),
        pltpu.VMEM((CH, DH, CW), jnp.float32),
        pltpu.SemaphoreType.DMA,
        pltpu.SemaphoreType.DMA,
    ],
)
def _sc_gather(in_t, ind_hbm, out_t, raw_v, idx_v, buf0, buf1, buf2, gsem, wsem):
    wid = lax.axis_index("s") * NC + lax.axis_index("c")
    c0 = wid * CW
    bufs = [buf0, buf1, buf2]

    pltpu.sync_copy(ind_hbm, raw_v.at[pl.ds(0, K)])
    for c in range(4):
        v = raw_v[pl.ds(c * 16, 16)]
        idx_v[c, pl.ds(0, 16)] = lax.rem(lax.rem(v, S) + S, S)

    def gather(t):
        c, h, dh, _, n = _TASKS[t]
        pltpu.make_async_copy(
            in_t.at[idx_v.at[c, pl.ds(h * CH, n)],
                    pl.ds(dh * DH, DH), pl.ds(c0, CW)],
            bufs[t % NBUF].at[pl.ds(0, n)],
            gsem,
        ).start()

    def wait_gather(t):
        _, _, _, _, n = _TASKS[t]
        pltpu.make_async_copy(
            in_t.at[pl.ds(0, n), pl.ds(0, DH), pl.ds(c0, CW)],
            bufs[t % NBUF].at[pl.ds(0, n)],
            gsem,
        ).wait()

    def write(t):
        _, _, dh, j0, n = _TASKS[t]
        pltpu.make_async_copy(
            bufs[t % NBUF].at[pl.ds(0, n)],
            out_t.at[pl.ds(j0, n), pl.ds(dh * DH, DH), pl.ds(c0, CW)],
            wsem,
        ).start()

    def wait_write(t):
        _, _, _, _, n = _TASKS[t]
        pltpu.make_async_copy(
            bufs[t % NBUF].at[pl.ds(0, n)],
            out_t.at[pl.ds(0, n), pl.ds(0, DH), pl.ds(c0, CW)],
            wsem,
        ).wait()

    T = len(_TASKS)
    for t in range(min(NBUF - 1, T)):
        gather(t)
    waited = 0
    for t in range(T):
        k = t + NBUF - 1
        if k < T:
            if k - NBUF >= 0:
                wait_write(k - NBUF)
                waited = k - NBUF + 1
            gather(k)
        wait_gather(t)
        write(t)
    for t in range(waited, T):
        wait_write(t)


def kernel(inputs, indices):
    in_t = jnp.transpose(inputs, (1, 2, 0))
    out_t = _sc_gather(in_t, indices.astype(jnp.int32))
    return jnp.transpose(out_t, (2, 0, 1))
